# baseline (speedup 1.0000x reference)
import jax
import jax.numpy as jnp
from jax.experimental import pallas as pl
from jax.experimental.pallas import tpu as pltpu


def _copy_body(in_ref, out_ref, sem):
    copy = pltpu.make_async_copy(in_ref, out_ref, sem)
    copy.start()
    copy.wait()


def kernel(_input):
    out = pl.pallas_call(
        _copy_body,
        in_specs=[pl.BlockSpec(memory_space=pl.ANY)],
        out_specs=pl.BlockSpec(memory_space=pl.ANY),
        out_shape=jax.ShapeDtypeStruct(_input.shape, _input.dtype),
        scratch_shapes=[pltpu.SemaphoreType.DMA],
    )(_input)
    return out


# speedup vs baseline: 9.9945x; 9.9945x over previous
import jax
import jax.numpy as jnp
from jax.experimental import pallas as pl
from jax.experimental.pallas import tpu as pltpu

_ROWS = 27648
_COLS = 1024
_BLOCK_ROWS = 1728


def _copy_body(in_ref, out_ref):
    out_ref[...] = in_ref[...]


def kernel(_input):
    x = _input.reshape(_ROWS, _COLS)
    out = pl.pallas_call(
        _copy_body,
        grid=(_ROWS // _BLOCK_ROWS,),
        in_specs=[pl.BlockSpec((_BLOCK_ROWS, _COLS), lambda i: (i, 0))],
        out_specs=pl.BlockSpec((_BLOCK_ROWS, _COLS), lambda i: (i, 0)),
        out_shape=jax.ShapeDtypeStruct((_ROWS, _COLS), _input.dtype),
        compiler_params=pltpu.CompilerParams(
            dimension_semantics=("arbitrary",),
        ),
    )(x)
    return out.reshape(_input.shape)
